# baseline (speedup 1.0000x reference)
import functools

import jax
import jax.numpy as jnp
from jax.experimental import pallas as pl
from jax.experimental.pallas import tpu as pltpu

D_MODEL = 4096
NUM_EXPERTS = 64
TOP_K = 8
BLOCK_T = 1024
K_SPLIT = 2
_IMAX = NUM_EXPERTS - 1


def _epilogue(logits, idx_ref, w_ref, usage_acc, first):
    iota = jax.lax.broadcasted_iota(jnp.int32, logits.shape, 1)
    riota = (_IMAX - iota).astype(jnp.float32)

    work = logits
    vals = []
    fidxs = []
    for _ in range(TOP_K):
        m = jnp.max(work, axis=-1, keepdims=True)
        eq = work == m
        fidxs.append(jnp.max(jnp.where(eq, riota, -1.0), axis=-1,
                             keepdims=True))
        vals.append(m)
        work = jnp.where(eq, -jnp.inf, work)
    v = jnp.concatenate(vals, axis=-1)
    fidx = jnp.concatenate(fidxs, axis=-1)
    idx_ref[...] = _IMAX - fidx.astype(jnp.int32)
    ev = jnp.exp(v - v[:, :1])
    w_ref[...] = ev / jnp.sum(ev, axis=-1, keepdims=True)

    p = jnp.exp(logits - v[:, :1])
    p = p / jnp.sum(p, axis=-1, keepdims=True)
    psum = jnp.sum(p, axis=0)[None, :]

    @pl.when(first)
    def _init():
        usage_acc[...] = jnp.zeros_like(usage_acc)

    usage_acc[...] += psum


def _router_block(x_ref, wt_ref, idx_ref, w_ref, aux_ref, logits_buf,
                  usage_acc, *, nblocks, ntokens):
    i = pl.program_id(0)
    k = pl.program_id(1)
    slot = jax.lax.rem(i, 2)

    @pl.when(jnp.logical_and(i < nblocks, k == 0))
    def _matmul0():
        logits_buf[slot] = jnp.dot(x_ref[...], wt_ref[...],
                                   preferred_element_type=jnp.float32)

    @pl.when(jnp.logical_and(i < nblocks, k == 1))
    def _matmul1():
        logits_buf[slot] += jnp.dot(x_ref[...], wt_ref[...],
                                    preferred_element_type=jnp.float32)

    @pl.when(jnp.logical_and(i > 0, k == 0))
    def _epi():
        _epilogue(logits_buf[1 - slot], idx_ref, w_ref, usage_acc, i == 1)

    @pl.when(jnp.logical_and(i == nblocks, k == 1))
    def _finalize():
        u = usage_acc[...] / ntokens - 1.0 / NUM_EXPERTS
        aux_ref[...] = jnp.sum(u * u).reshape(1, 1)


def kernel(x, W):
    B, S, D = x.shape
    ntokens = B * S
    x2 = x.reshape(ntokens, D)
    wt = W.T
    nblocks = ntokens // BLOCK_T
    dk = D // K_SPLIT

    body = functools.partial(_router_block, nblocks=nblocks, ntokens=ntokens)
    idx, w, aux = pl.pallas_call(
        body,
        grid=(nblocks + 1, K_SPLIT),
        in_specs=[
            pl.BlockSpec((BLOCK_T, dk),
                         lambda i, k: (jnp.minimum(i, nblocks - 1), k)),
            pl.BlockSpec((dk, NUM_EXPERTS), lambda i, k: (k, 0)),
        ],
        out_specs=[
            pl.BlockSpec((BLOCK_T, TOP_K),
                         lambda i, k: (jnp.maximum(i - 1, 0), 0)),
            pl.BlockSpec((BLOCK_T, TOP_K),
                         lambda i, k: (jnp.maximum(i - 1, 0), 0)),
            pl.BlockSpec((1, 1), lambda i, k: (0, 0)),
        ],
        out_shape=[
            jax.ShapeDtypeStruct((ntokens, TOP_K), jnp.int32),
            jax.ShapeDtypeStruct((ntokens, TOP_K), jnp.float32),
            jax.ShapeDtypeStruct((1, 1), jnp.float32),
        ],
        scratch_shapes=[
            pltpu.VMEM((2, BLOCK_T, NUM_EXPERTS), jnp.float32),
            pltpu.VMEM((1, NUM_EXPERTS), jnp.float32),
        ],
    )(x2, wt)

    return (idx.reshape(B, S, TOP_K), w.reshape(B, S, TOP_K),
            aux.reshape(()))


# speedup vs baseline: 1.3211x; 1.3211x over previous
import functools

import jax
import jax.numpy as jnp
from jax.experimental import pallas as pl
from jax.experimental.pallas import tpu as pltpu

D_MODEL = 4096
NUM_EXPERTS = 64
TOP_K = 8
BLOCK_T = 1024
_IMAX = NUM_EXPERTS - 1


def _epilogue(logits, idx_ref, w_ref, usage_acc, first):
    iota = jax.lax.broadcasted_iota(jnp.int32, logits.shape, 1)
    riota = (_IMAX - iota).astype(jnp.float32)

    work = logits
    vals = []
    fidxs = []
    for _ in range(TOP_K):
        m = jnp.max(work, axis=-1, keepdims=True)
        eq = work == m
        fidxs.append(jnp.max(jnp.where(eq, riota, -1.0), axis=-1,
                             keepdims=True))
        vals.append(m)
        work = jnp.where(eq, -jnp.inf, work)
    v = jnp.concatenate(vals, axis=-1)
    fidx = jnp.concatenate(fidxs, axis=-1)
    idx_ref[...] = _IMAX - fidx.astype(jnp.int32)
    ev = jnp.exp(v - v[:, :1])
    w_ref[...] = ev / jnp.sum(ev, axis=-1, keepdims=True)

    p = jnp.exp(logits - v[:, :1])
    p = p / jnp.sum(p, axis=-1, keepdims=True)
    psum = jnp.sum(p, axis=0)[None, :]

    @pl.when(first)
    def _init():
        usage_acc[...] = jnp.zeros_like(usage_acc)

    usage_acc[...] += psum


def _router_block(x_ref, wt_ref, idx_ref, w_ref, aux_ref, logits_buf,
                  usage_acc, *, nblocks, ntokens):
    i = pl.program_id(0)
    slot = jax.lax.rem(i, 2)

    @pl.when(i < nblocks)
    def _matmul():
        logits_buf[slot] = jnp.dot(x_ref[...], wt_ref[...],
                                   preferred_element_type=jnp.float32)

    @pl.when(i > 0)
    def _epi():
        _epilogue(logits_buf[1 - slot], idx_ref, w_ref, usage_acc, i == 1)

    @pl.when(i == nblocks)
    def _finalize():
        u = usage_acc[...] / ntokens - 1.0 / NUM_EXPERTS
        aux_ref[...] = jnp.sum(u * u).reshape(1, 1)


def kernel(x, W):
    B, S, D = x.shape
    ntokens = B * S
    x2 = x.reshape(ntokens, D)
    wt = W.T
    nblocks = ntokens // BLOCK_T

    body = functools.partial(_router_block, nblocks=nblocks, ntokens=ntokens)
    idx, w, aux = pl.pallas_call(
        body,
        grid=(nblocks + 1,),
        in_specs=[
            pl.BlockSpec((BLOCK_T, D),
                         lambda i: (jnp.minimum(i, nblocks - 1), 0)),
            pl.BlockSpec((D, NUM_EXPERTS), lambda i: (0, 0)),
        ],
        out_specs=[
            pl.BlockSpec((BLOCK_T, TOP_K),
                         lambda i: (jnp.maximum(i - 1, 0), 0)),
            pl.BlockSpec((BLOCK_T, TOP_K),
                         lambda i: (jnp.maximum(i - 1, 0), 0)),
            pl.BlockSpec((1, 1), lambda i: (0, 0)),
        ],
        out_shape=[
            jax.ShapeDtypeStruct((ntokens, TOP_K), jnp.int32),
            jax.ShapeDtypeStruct((ntokens, TOP_K), jnp.float32),
            jax.ShapeDtypeStruct((1, 1), jnp.float32),
        ],
        scratch_shapes=[
            pltpu.VMEM((2, BLOCK_T, NUM_EXPERTS), jnp.float32),
            pltpu.VMEM((1, NUM_EXPERTS), jnp.float32),
        ],
    )(x2, wt)

    return (idx.reshape(B, S, TOP_K), w.reshape(B, S, TOP_K),
            aux.reshape(()))


# speedup vs baseline: 1.3846x; 1.0481x over previous
import functools

import jax
import jax.numpy as jnp
from jax.experimental import pallas as pl
from jax.experimental.pallas import tpu as pltpu

D_MODEL = 4096
NUM_EXPERTS = 64
TOP_K = 8
BLOCK_T = 1024
NBUF = 3
_IMAX = NUM_EXPERTS - 1


def _epilogue(logits, idx_ref, w_ref, usage_acc, first):
    iota = jax.lax.broadcasted_iota(jnp.int32, logits.shape, 1)
    riota = (_IMAX - iota).astype(jnp.float32)

    work = logits
    vals = []
    fidxs = []
    for _ in range(TOP_K):
        m = jnp.max(work, axis=-1, keepdims=True)
        eq = work == m
        fidxs.append(jnp.max(jnp.where(eq, riota, -1.0), axis=-1,
                             keepdims=True))
        vals.append(m)
        work = jnp.where(eq, -jnp.inf, work)
    v = jnp.concatenate(vals, axis=-1)
    fidx = jnp.concatenate(fidxs, axis=-1)
    idx_ref[...] = _IMAX - fidx.astype(jnp.int32)
    ev = jnp.exp(v - v[:, :1])
    w_ref[...] = ev / jnp.sum(ev, axis=-1, keepdims=True)

    p = jnp.exp(logits - v[:, :1])
    p = p / jnp.sum(p, axis=-1, keepdims=True)
    psum = jnp.sum(p, axis=0)[None, :]

    @pl.when(first)
    def _init():
        usage_acc[...] = jnp.zeros_like(usage_acc)

    usage_acc[...] += psum


def _router_block(x_hbm, wt_ref, idx_ref, w_ref, aux_ref, xring,
                  logits_buf, usage_acc, sems, *, nblocks, ntokens):
    i = pl.program_id(0)
    slot = jax.lax.rem(i, 2)

    def _copy(b):
        s = jax.lax.rem(b, NBUF)
        return pltpu.make_async_copy(
            x_hbm.at[pl.ds(b * BLOCK_T, BLOCK_T), :], xring.at[s],
            sems.at[s])

    @pl.when(i == 0)
    def _prime():
        _copy(0).start()
        _copy(1).start()
        _copy(2).start()

    @pl.when(jnp.logical_and(i > 0, i + 2 < nblocks))
    def _lookahead():
        _copy(i + 2).start()

    @pl.when(i < nblocks)
    def _matmul():
        _copy(i).wait()
        logits_buf[slot] = jnp.dot(xring[jax.lax.rem(i, NBUF)], wt_ref[...],
                                   preferred_element_type=jnp.float32)

    @pl.when(i > 0)
    def _epi():
        _epilogue(logits_buf[1 - slot], idx_ref, w_ref, usage_acc, i == 1)

    @pl.when(i == nblocks)
    def _finalize():
        u = usage_acc[...] / ntokens - 1.0 / NUM_EXPERTS
        aux_ref[...] = jnp.sum(u * u).reshape(1, 1)


def kernel(x, W):
    B, S, D = x.shape
    ntokens = B * S
    x2 = x.reshape(ntokens, D)
    wt = W.T
    nblocks = ntokens // BLOCK_T

    body = functools.partial(_router_block, nblocks=nblocks, ntokens=ntokens)
    idx, w, aux = pl.pallas_call(
        body,
        grid=(nblocks + 1,),
        in_specs=[
            pl.BlockSpec(memory_space=pl.ANY),
            pl.BlockSpec((D, NUM_EXPERTS), lambda i: (0, 0)),
        ],
        out_specs=[
            pl.BlockSpec((BLOCK_T, TOP_K),
                         lambda i: (jnp.maximum(i - 1, 0), 0)),
            pl.BlockSpec((BLOCK_T, TOP_K),
                         lambda i: (jnp.maximum(i - 1, 0), 0)),
            pl.BlockSpec((1, 1), lambda i: (0, 0)),
        ],
        out_shape=[
            jax.ShapeDtypeStruct((ntokens, TOP_K), jnp.int32),
            jax.ShapeDtypeStruct((ntokens, TOP_K), jnp.float32),
            jax.ShapeDtypeStruct((1, 1), jnp.float32),
        ],
        scratch_shapes=[
            pltpu.VMEM((NBUF, BLOCK_T, D_MODEL), jnp.float32),
            pltpu.VMEM((2, BLOCK_T, NUM_EXPERTS), jnp.float32),
            pltpu.VMEM((1, NUM_EXPERTS), jnp.float32),
            pltpu.SemaphoreType.DMA((NBUF,)),
        ],
    )(x2, wt)

    return (idx.reshape(B, S, TOP_K), w.reshape(B, S, TOP_K),
            aux.reshape(()))
